# baseline (speedup 1.0000x reference)
import functools

import jax
import jax.numpy as jnp
from jax.experimental import pallas as pl
from jax.experimental.pallas import tpu as pltpu

_BT = 1024
_LANES = 128


def _fwd_kernel(maj_ref, car_ref, gpa_ref, int0_ref, int1_ref, crs0_ref,
                crs1_ref, mtab_ref, ctab_ref, wint_ref, bint_ref, wcrs_ref,
                bcrs_ref, w1_ref, b1_ref, w2_ref, b2_ref, w3_ref, b3_ref,
                out_ref):
    bt = int0_ref.shape[0]
    n_maj = mtab_ref.shape[0]
    n_car = ctab_ref.shape[0]
    f32 = jnp.float32
    bf16 = jnp.bfloat16
    ni_h = int0_ref.shape[1]
    nc_h = crs0_ref.shape[1]

    wint = wint_ref[...].astype(bf16)
    ie = jnp.dot(int0_ref[...].astype(bf16), wint[0:ni_h, :],
                 preferred_element_type=f32)
    ie += jnp.dot(int1_ref[...].astype(bf16), wint[ni_h:2 * ni_h, :],
                  preferred_element_type=f32)
    ie = jnp.maximum(ie + bint_ref[...], 0.0)

    wcrs = wcrs_ref[...].astype(bf16)
    ce = jnp.dot(crs0_ref[...].astype(bf16), wcrs[0:nc_h, :],
                 preferred_element_type=f32)
    ce += jnp.dot(crs1_ref[...].astype(bf16), wcrs[nc_h:2 * nc_h, :],
                  preferred_element_type=f32)
    ce = jnp.maximum(ce + bcrs_ref[...], 0.0)

    ngrp = maj_ref.shape[2]
    majt = maj_ref[0]
    cart = car_ref[0]
    gpat = gpa_ref[0]
    iota_maj = jax.lax.broadcasted_iota(jnp.int32, (_LANES, n_maj), 1)
    iota_car = jax.lax.broadcasted_iota(jnp.int32, (_LANES, n_car), 1)
    maj_oh = jnp.concatenate(
        [(majt[:, s:s + 1] == iota_maj).astype(bf16) for s in range(ngrp)],
        axis=0)
    car_oh = jnp.concatenate(
        [(cart[:, s:s + 1] == iota_car).astype(bf16) for s in range(ngrp)],
        axis=0)
    me = jnp.dot(maj_oh, mtab_ref[...].astype(bf16), preferred_element_type=f32)
    cae = jnp.dot(car_oh, ctab_ref[...].astype(bf16), preferred_element_type=f32)

    gpa2 = jnp.concatenate(
        [gpat[:, s:s + 1] for s in range(ngrp)], axis=0)
    gpa_n = (gpa2 - 3.0) * (1.0 / jnp.sqrt(jnp.float32(0.25 + 1e-6)))

    d = mtab_ref.shape[1]
    w1b = w1_ref[...].astype(bf16)
    packed = jnp.concatenate(
        [me.astype(bf16), cae.astype(bf16), ie.astype(bf16), ce.astype(bf16)],
        axis=1)
    h = jnp.dot(packed, w1b[0:4 * d, :], preferred_element_type=f32)
    h += gpa_n * w1_ref[4 * d:4 * d + 1, :]
    h = jnp.maximum(h + b1_ref[...], 0.0)

    h = jnp.maximum(
        jnp.dot(h.astype(bf16), w2_ref[...].astype(bf16),
                preferred_element_type=f32) + b2_ref[...], 0.0)
    out_ref[...] = (
        jnp.dot(h.astype(bf16), w3_ref[...].astype(bf16),
                preferred_element_type=f32) + b3_ref[...])


@functools.partial(jax.jit, static_argnames=())
def kernel(major, career_goal, interests, completed_courses, gpa,
           major_table, career_table, W_int, b_int, W_crs, b_crs,
           W1, b1, W2, b2, W3, b3):
    B, NI = interests.shape
    NC = completed_courses.shape[1]
    D = major_table.shape[1]
    N_MAJ = major_table.shape[0]
    N_CAR = career_table.shape[0]
    OUT = W3.shape[1]
    H1 = W1.shape[1]
    H2 = W2.shape[1]

    L = _LANES
    bt = _BT
    grid = (B // bt,)
    gsub = bt // L

    def to_t(x):
        return x.reshape(B // bt, gsub, L).transpose(0, 2, 1)

    majc = to_t(major)
    carc = to_t(career_goal)
    gpac = to_t(gpa)

    def row(i):
        return (i, 0)

    def col1(i):
        return (i, 1)

    def rep(i):
        return (0, 0)

    out = pl.pallas_call(
        _fwd_kernel,
        grid=grid,
        in_specs=[
            pl.BlockSpec((1, L, gsub), lambda i: (i, 0, 0)),
            pl.BlockSpec((1, L, gsub), lambda i: (i, 0, 0)),
            pl.BlockSpec((1, L, gsub), lambda i: (i, 0, 0)),
            pl.BlockSpec((bt, NI // 2), row),
            pl.BlockSpec((bt, NI // 2), col1),
            pl.BlockSpec((bt, NC // 2), row),
            pl.BlockSpec((bt, NC // 2), col1),
            pl.BlockSpec((N_MAJ, D), rep),
            pl.BlockSpec((N_CAR, D), rep),
            pl.BlockSpec((NI, D), rep),
            pl.BlockSpec((1, D), rep),
            pl.BlockSpec((NC, D), rep),
            pl.BlockSpec((1, D), rep),
            pl.BlockSpec((4 * D + 1, H1), rep),
            pl.BlockSpec((1, H1), rep),
            pl.BlockSpec((H1, H2), rep),
            pl.BlockSpec((1, H2), rep),
            pl.BlockSpec((H2, OUT), rep),
            pl.BlockSpec((1, OUT), rep),
        ],
        out_specs=pl.BlockSpec((bt, OUT), row),
        out_shape=jax.ShapeDtypeStruct((B, OUT), jnp.float32),
        compiler_params=pltpu.CompilerParams(
            dimension_semantics=("parallel",)),
    )(majc, carc, gpac, interests, interests, completed_courses,
      completed_courses, major_table, career_table, W_int,
      b_int.reshape(1, D), W_crs, b_crs.reshape(1, D), W1, b1.reshape(1, H1),
      W2, b2.reshape(1, H2), W3, b3.reshape(1, OUT))
    return out


# speedup vs baseline: 1.1618x; 1.1618x over previous
import functools

import jax
import jax.numpy as jnp
from jax.experimental import pallas as pl
from jax.experimental.pallas import tpu as pltpu

_BT = 4096
_LANES = 128


def _fwd_kernel(maj_ref, car_ref, gpa_ref, int0_ref, int1_ref, crs0_ref,
                crs1_ref, mtab_ref, ctab_ref, wint_ref, bint_ref, wcrs_ref,
                bcrs_ref, w1_ref, b1_ref, w2_ref, b2_ref, w3_ref, b3_ref,
                out_ref):
    bt = int0_ref.shape[0]
    n_maj = mtab_ref.shape[0]
    n_car = ctab_ref.shape[0]
    f32 = jnp.float32
    bf16 = jnp.bfloat16
    ni_h = int0_ref.shape[1]
    nc_h = crs0_ref.shape[1]

    wint = wint_ref[...].astype(bf16)
    ie = jnp.dot(int0_ref[...].astype(bf16), wint[0:ni_h, :],
                 preferred_element_type=f32)
    ie += jnp.dot(int1_ref[...].astype(bf16), wint[ni_h:2 * ni_h, :],
                  preferred_element_type=f32)
    ie = jnp.maximum(ie + bint_ref[...], 0.0)

    wcrs = wcrs_ref[...].astype(bf16)
    ce = jnp.dot(crs0_ref[...].astype(bf16), wcrs[0:nc_h, :],
                 preferred_element_type=f32)
    ce += jnp.dot(crs1_ref[...].astype(bf16), wcrs[nc_h:2 * nc_h, :],
                  preferred_element_type=f32)
    ce = jnp.maximum(ce + bcrs_ref[...], 0.0)

    ngrp = maj_ref.shape[2]
    majt = maj_ref[0]
    cart = car_ref[0]
    gpat = gpa_ref[0]
    iota_maj = jax.lax.broadcasted_iota(jnp.int32, (_LANES, n_maj), 1)
    iota_car = jax.lax.broadcasted_iota(jnp.int32, (_LANES, n_car), 1)
    maj_oh = jnp.concatenate(
        [(majt[:, s:s + 1] == iota_maj).astype(bf16) for s in range(ngrp)],
        axis=0)
    car_oh = jnp.concatenate(
        [(cart[:, s:s + 1] == iota_car).astype(bf16) for s in range(ngrp)],
        axis=0)
    me = jnp.dot(maj_oh, mtab_ref[...].astype(bf16), preferred_element_type=f32)
    cae = jnp.dot(car_oh, ctab_ref[...].astype(bf16), preferred_element_type=f32)

    gpa2 = jnp.concatenate(
        [gpat[:, s:s + 1] for s in range(ngrp)], axis=0)
    gpa_n = (gpa2 - 3.0) * (1.0 / jnp.sqrt(jnp.float32(0.25 + 1e-6)))

    d = mtab_ref.shape[1]
    w1b = w1_ref[...].astype(bf16)
    packed = jnp.concatenate(
        [me.astype(bf16), cae.astype(bf16), ie.astype(bf16), ce.astype(bf16)],
        axis=1)
    h = jnp.dot(packed, w1b[0:4 * d, :], preferred_element_type=f32)
    h += gpa_n * w1_ref[4 * d:4 * d + 1, :]
    h = jnp.maximum(h + b1_ref[...], 0.0)

    h = jnp.maximum(
        jnp.dot(h.astype(bf16), w2_ref[...].astype(bf16),
                preferred_element_type=f32) + b2_ref[...], 0.0)
    out_ref[...] = (
        jnp.dot(h.astype(bf16), w3_ref[...].astype(bf16),
                preferred_element_type=f32) + b3_ref[...])


@functools.partial(jax.jit, static_argnames=())
def kernel(major, career_goal, interests, completed_courses, gpa,
           major_table, career_table, W_int, b_int, W_crs, b_crs,
           W1, b1, W2, b2, W3, b3):
    B, NI = interests.shape
    NC = completed_courses.shape[1]
    D = major_table.shape[1]
    N_MAJ = major_table.shape[0]
    N_CAR = career_table.shape[0]
    OUT = W3.shape[1]
    H1 = W1.shape[1]
    H2 = W2.shape[1]

    L = _LANES
    bt = _BT
    grid = (B // bt,)
    gsub = bt // L

    def to_t(x):
        return x.reshape(B // bt, gsub, L).transpose(0, 2, 1)

    majc = to_t(major)
    carc = to_t(career_goal)
    gpac = to_t(gpa)

    def row(i):
        return (i, 0)

    def col1(i):
        return (i, 1)

    def rep(i):
        return (0, 0)

    out = pl.pallas_call(
        _fwd_kernel,
        grid=grid,
        in_specs=[
            pl.BlockSpec((1, L, gsub), lambda i: (i, 0, 0)),
            pl.BlockSpec((1, L, gsub), lambda i: (i, 0, 0)),
            pl.BlockSpec((1, L, gsub), lambda i: (i, 0, 0)),
            pl.BlockSpec((bt, NI // 2), row),
            pl.BlockSpec((bt, NI // 2), col1),
            pl.BlockSpec((bt, NC // 2), row),
            pl.BlockSpec((bt, NC // 2), col1),
            pl.BlockSpec((N_MAJ, D), rep),
            pl.BlockSpec((N_CAR, D), rep),
            pl.BlockSpec((NI, D), rep),
            pl.BlockSpec((1, D), rep),
            pl.BlockSpec((NC, D), rep),
            pl.BlockSpec((1, D), rep),
            pl.BlockSpec((4 * D + 1, H1), rep),
            pl.BlockSpec((1, H1), rep),
            pl.BlockSpec((H1, H2), rep),
            pl.BlockSpec((1, H2), rep),
            pl.BlockSpec((H2, OUT), rep),
            pl.BlockSpec((1, OUT), rep),
        ],
        out_specs=pl.BlockSpec((bt, OUT), row),
        out_shape=jax.ShapeDtypeStruct((B, OUT), jnp.float32),
        compiler_params=pltpu.CompilerParams(
            dimension_semantics=("parallel",)),
    )(majc, carc, gpac, interests, interests, completed_courses,
      completed_courses, major_table, career_table, W_int,
      b_int.reshape(1, D), W_crs, b_crs.reshape(1, D), W1, b1.reshape(1, H1),
      W2, b2.reshape(1, H2), W3, b3.reshape(1, OUT))
    return out


# speedup vs baseline: 1.1618x; 1.0001x over previous
import functools

import jax
import jax.numpy as jnp
from jax.experimental import pallas as pl
from jax.experimental.pallas import tpu as pltpu

_BT = 4096
_LANES = 128


def _fwd_kernel(maj_ref, car_ref, gpa_ref, int0_ref, int1_ref, crs0_ref,
                crs1_ref, mtab_ref, ctab_ref, wint_ref, bint_ref, wcrs_ref,
                bcrs_ref, w1_ref, b1_ref, w2_ref, b2_ref, w3_ref, b3_ref,
                out_ref):
    bt = int0_ref.shape[0]
    n_maj = mtab_ref.shape[0]
    n_car = ctab_ref.shape[0]
    f32 = jnp.float32
    bf16 = jnp.bfloat16
    ni_h = int0_ref.shape[1]
    nc_h = crs0_ref.shape[1]

    wint = wint_ref[...].astype(bf16)
    ie = jnp.dot(int0_ref[...].astype(bf16), wint[0:ni_h, :],
                 preferred_element_type=f32)
    ie += jnp.dot(int1_ref[...].astype(bf16), wint[ni_h:2 * ni_h, :],
                  preferred_element_type=f32)
    ie = jnp.maximum(ie + bint_ref[...], 0.0).astype(bf16)

    wcrs = wcrs_ref[...].astype(bf16)
    ce = jnp.dot(crs0_ref[...].astype(bf16), wcrs[0:nc_h, :],
                 preferred_element_type=f32)
    ce += jnp.dot(crs1_ref[...].astype(bf16), wcrs[nc_h:2 * nc_h, :],
                  preferred_element_type=f32)
    ce = jnp.maximum(ce + bcrs_ref[...], 0.0).astype(bf16)

    ngrp = maj_ref.shape[2]
    majt = maj_ref[0]
    cart = car_ref[0]
    gpat = gpa_ref[0]
    iota_maj = jax.lax.broadcasted_iota(jnp.int32, (_LANES, n_maj), 1)
    iota_car = jax.lax.broadcasted_iota(jnp.int32, (_LANES, n_car), 1)
    maj_oh = jnp.concatenate(
        [(majt[:, s:s + 1] == iota_maj).astype(bf16) for s in range(ngrp)],
        axis=0)
    car_oh = jnp.concatenate(
        [(cart[:, s:s + 1] == iota_car).astype(bf16) for s in range(ngrp)],
        axis=0)
    me = jnp.dot(maj_oh, mtab_ref[...].astype(bf16),
                 preferred_element_type=f32).astype(bf16)
    cae = jnp.dot(car_oh, ctab_ref[...].astype(bf16),
                  preferred_element_type=f32).astype(bf16)

    gpa2 = jnp.concatenate(
        [gpat[:, s:s + 1] for s in range(ngrp)], axis=0)
    gpa_n = (gpa2 - 3.0) * (1.0 / jnp.sqrt(jnp.float32(0.25 + 1e-6)))

    d = mtab_ref.shape[1]
    w1b = w1_ref[...].astype(bf16)
    packed = jnp.concatenate([me, cae, ie, ce], axis=1)
    h = jnp.dot(packed, w1b[0:4 * d, :], preferred_element_type=f32)
    h += gpa_n * w1_ref[4 * d:4 * d + 1, :]
    h = jnp.maximum(h + b1_ref[...], 0.0).astype(bf16)

    h = jnp.maximum(
        jnp.dot(h, w2_ref[...].astype(bf16),
                preferred_element_type=f32) + b2_ref[...], 0.0).astype(bf16)
    out_ref[...] = (
        jnp.dot(h, w3_ref[...].astype(bf16),
                preferred_element_type=f32) + b3_ref[...])


@functools.partial(jax.jit, static_argnames=())
def kernel(major, career_goal, interests, completed_courses, gpa,
           major_table, career_table, W_int, b_int, W_crs, b_crs,
           W1, b1, W2, b2, W3, b3):
    B, NI = interests.shape
    NC = completed_courses.shape[1]
    D = major_table.shape[1]
    N_MAJ = major_table.shape[0]
    N_CAR = career_table.shape[0]
    OUT = W3.shape[1]
    H1 = W1.shape[1]
    H2 = W2.shape[1]

    L = _LANES
    bt = _BT
    grid = (B // bt,)
    gsub = bt // L

    def to_t(x):
        return x.reshape(B // bt, gsub, L).transpose(0, 2, 1)

    majc = to_t(major)
    carc = to_t(career_goal)
    gpac = to_t(gpa)

    def row(i):
        return (i, 0)

    def col1(i):
        return (i, 1)

    def rep(i):
        return (0, 0)

    out = pl.pallas_call(
        _fwd_kernel,
        grid=grid,
        in_specs=[
            pl.BlockSpec((1, L, gsub), lambda i: (i, 0, 0)),
            pl.BlockSpec((1, L, gsub), lambda i: (i, 0, 0)),
            pl.BlockSpec((1, L, gsub), lambda i: (i, 0, 0)),
            pl.BlockSpec((bt, NI // 2), row),
            pl.BlockSpec((bt, NI // 2), col1),
            pl.BlockSpec((bt, NC // 2), row),
            pl.BlockSpec((bt, NC // 2), col1),
            pl.BlockSpec((N_MAJ, D), rep),
            pl.BlockSpec((N_CAR, D), rep),
            pl.BlockSpec((NI, D), rep),
            pl.BlockSpec((1, D), rep),
            pl.BlockSpec((NC, D), rep),
            pl.BlockSpec((1, D), rep),
            pl.BlockSpec((4 * D + 1, H1), rep),
            pl.BlockSpec((1, H1), rep),
            pl.BlockSpec((H1, H2), rep),
            pl.BlockSpec((1, H2), rep),
            pl.BlockSpec((H2, OUT), rep),
            pl.BlockSpec((1, OUT), rep),
        ],
        out_specs=pl.BlockSpec((bt, OUT), row),
        out_shape=jax.ShapeDtypeStruct((B, OUT), jnp.float32),
        compiler_params=pltpu.CompilerParams(
            dimension_semantics=("parallel",)),
    )(majc, carc, gpac, interests, interests, completed_courses,
      completed_courses, major_table, career_table, W_int,
      b_int.reshape(1, D), W_crs, b_crs.reshape(1, D), W1, b1.reshape(1, H1),
      W2, b2.reshape(1, H2), W3, b3.reshape(1, OUT))
    return out


# speedup vs baseline: 1.1989x; 1.0319x over previous
import functools

import jax
import jax.numpy as jnp
from jax.experimental import pallas as pl
from jax.experimental.pallas import tpu as pltpu

_BT = 2048
_LANES = 128


def _make_body(B, NI, NC):
    bt = _BT
    nt = B // bt
    gsub = bt // _LANES

    def body(majt_ref, cart_ref, gpat_ref, int_hbm, crs_hbm, mtab_ref,
             ctab_ref, wint_ref, bint_ref, wcrs_ref, bcrs_ref, w1_ref,
             b1_ref, w2_ref, b2_ref, w3_ref, b3_ref, out_ref,
             ibuf0, ibuf1, cbuf0, cbuf1, si0, si1, sc0, sc1):
        f32 = jnp.float32
        bf16 = jnp.bfloat16
        n_maj = mtab_ref.shape[0]
        n_car = ctab_ref.shape[0]
        d = mtab_ref.shape[1]
        ibufs, cbufs = (ibuf0, ibuf1), (cbuf0, cbuf1)
        isems, csems = (si0, si1), (sc0, sc1)

        def start(t):
            s = t % 2
            pltpu.make_async_copy(int_hbm.at[pl.ds(t * bt, bt), :],
                                  ibufs[s], isems[s]).start()
            pltpu.make_async_copy(crs_hbm.at[pl.ds(t * bt, bt), :],
                                  cbufs[s], csems[s]).start()

        def wait(t):
            s = t % 2
            pltpu.make_async_copy(int_hbm.at[pl.ds(t * bt, bt), :],
                                  ibufs[s], isems[s]).wait()
            pltpu.make_async_copy(crs_hbm.at[pl.ds(t * bt, bt), :],
                                  cbufs[s], csems[s]).wait()

        wint = wint_ref[...].astype(bf16)
        wcrs = wcrs_ref[...].astype(bf16)
        mtab = mtab_ref[...].astype(bf16)
        ctab = ctab_ref[...].astype(bf16)
        w1b = w1_ref[...].astype(bf16)
        w2b = w2_ref[...].astype(bf16)
        w3b = w3_ref[...].astype(bf16)
        iota_maj = jax.lax.broadcasted_iota(jnp.int32, (_LANES, n_maj), 1)
        iota_car = jax.lax.broadcasted_iota(jnp.int32, (_LANES, n_car), 1)

        start(0)
        for t in range(nt):
            if t + 1 < nt:
                start(t + 1)
            wait(t)
            s = t % 2

            ie = jnp.dot(ibufs[s][...].astype(bf16), wint,
                         preferred_element_type=f32)
            ie = jnp.maximum(ie + bint_ref[...], 0.0).astype(bf16)
            ce = jnp.dot(cbufs[s][...].astype(bf16), wcrs,
                         preferred_element_type=f32)
            ce = jnp.maximum(ce + bcrs_ref[...], 0.0).astype(bf16)

            g0 = t * gsub
            maj_oh = jnp.concatenate(
                [(majt_ref[:, g:g + 1] == iota_maj).astype(bf16)
                 for g in range(g0, g0 + gsub)], axis=0)
            car_oh = jnp.concatenate(
                [(cart_ref[:, g:g + 1] == iota_car).astype(bf16)
                 for g in range(g0, g0 + gsub)], axis=0)
            me = jnp.dot(maj_oh, mtab,
                         preferred_element_type=f32).astype(bf16)
            cae = jnp.dot(car_oh, ctab,
                          preferred_element_type=f32).astype(bf16)

            gpa2 = jnp.concatenate(
                [gpat_ref[:, g:g + 1] for g in range(g0, g0 + gsub)], axis=0)
            gpa_n = (gpa2 - 3.0) * (1.0 / jnp.sqrt(jnp.float32(0.25 + 1e-6)))

            packed = jnp.concatenate([me, cae, ie, ce], axis=1)
            h = jnp.dot(packed, w1b[0:4 * d, :], preferred_element_type=f32)
            h += gpa_n * w1_ref[4 * d:4 * d + 1, :]
            h = jnp.maximum(h + b1_ref[...], 0.0).astype(bf16)
            h = jnp.maximum(
                jnp.dot(h, w2b, preferred_element_type=f32)
                + b2_ref[...], 0.0).astype(bf16)
            out_ref[pl.ds(t * bt, bt), :] = (
                jnp.dot(h, w3b, preferred_element_type=f32) + b3_ref[...])

    return body


@functools.partial(jax.jit, static_argnames=())
def kernel(major, career_goal, interests, completed_courses, gpa,
           major_table, career_table, W_int, b_int, W_crs, b_crs,
           W1, b1, W2, b2, W3, b3):
    B, NI = interests.shape
    NC = completed_courses.shape[1]
    D = major_table.shape[1]
    N_MAJ = major_table.shape[0]
    N_CAR = career_table.shape[0]
    OUT = W3.shape[1]
    H1 = W1.shape[1]
    H2 = W2.shape[1]
    L = _LANES
    bt = _BT
    f32 = jnp.float32

    def to_t(x):
        return x.reshape(B // L, L).swapaxes(0, 1)

    vmem = functools.partial(pl.BlockSpec, memory_space=pltpu.VMEM)
    hbm = functools.partial(pl.BlockSpec, memory_space=pl.ANY)

    out = pl.pallas_call(
        _make_body(B, NI, NC),
        in_specs=[
            vmem(), vmem(), vmem(),
            hbm(), hbm(),
            vmem(), vmem(),
            vmem(), vmem(), vmem(), vmem(),
            vmem(), vmem(), vmem(), vmem(), vmem(), vmem(),
        ],
        out_specs=vmem(),
        out_shape=jax.ShapeDtypeStruct((B, OUT), f32),
        scratch_shapes=[
            pltpu.VMEM((bt, NI), f32), pltpu.VMEM((bt, NI), f32),
            pltpu.VMEM((bt, NC), f32), pltpu.VMEM((bt, NC), f32),
            pltpu.SemaphoreType.DMA, pltpu.SemaphoreType.DMA,
            pltpu.SemaphoreType.DMA, pltpu.SemaphoreType.DMA,
        ],
    )(to_t(major), to_t(career_goal), to_t(gpa),
      interests, completed_courses, major_table, career_table,
      W_int, b_int.reshape(1, D), W_crs, b_crs.reshape(1, D),
      W1, b1.reshape(1, H1), W2, b2.reshape(1, H2), W3, b3.reshape(1, OUT))
    return out
